# Initial kernel scaffold; baseline (speedup 1.0000x reference)
import jax
import jax.numpy as jnp
from jax.experimental import pallas as pl


def kernel(x, edge_index, W, b, alpha):
    raise NotImplementedError("write your pallas kernel here")


# speedup vs baseline: 32.1282x; 32.1282x over previous
import functools

import jax
import jax.numpy as jnp
from jax import lax
from jax.experimental import pallas as pl
from jax.experimental.pallas import tpu as pltpu
from jax.experimental.pallas import tpu_sc as plsc

N_NODES = 10000
N_EDGES = 320000
D = 128

NC, NS = 2, 16
NW = NC * NS
CH = 125
EPT = N_EDGES // NW
NCHUNK = EPT // CH
NROWCH = N_EDGES // CH
NBINS = 10240
BPT = NBINS // NS
NPAD = 10240
RPT = NPAD // NS
RCH = 128

_f32 = jnp.float32

_mesh = plsc.VectorSubcoreMesh(core_axis_name="c", subcore_axis_name="s")


@functools.partial(
    pl.kernel,
    out_type=jax.ShapeDtypeStruct((NC, NBINS), _f32),
    mesh=_mesh,
    scratch_types=[
        pltpu.VMEM((NCHUNK, CH), jnp.int32),
        pltpu.VMEM((128,), _f32),
        pltpu.VMEM((BPT,), _f32),
        pltpu.VMEM_SHARED((NBINS,), _f32),
    ],
)
def _deg_call(dst2, degp, idxv, ones_v, buf, deg_sh):
    c = lax.axis_index("c")
    s = lax.axis_index("s")
    w = c * NS + s

    for i in range(8):
        ones_v[pl.ds(i * 16, 16)] = jnp.ones((16,), _f32)

    def _z(i, _):
        buf[pl.ds(i * 16, 16)] = jnp.zeros((16,), _f32)
        return 0

    lax.fori_loop(0, BPT // 16, _z, 0)
    pltpu.sync_copy(buf, deg_sh.at[pl.ds(s * BPT, BPT)])
    plsc.subcore_barrier()

    pltpu.sync_copy(dst2.at[pl.ds(w * NCHUNK, NCHUNK)], idxv)

    def _scatter(j, _):
        pltpu.sync_copy(ones_v.at[pl.ds(0, CH)], deg_sh.at[idxv.at[j]],
                        add=True)
        return 0

    lax.fori_loop(0, NCHUNK, _scatter, 0)
    plsc.subcore_barrier()

    pltpu.sync_copy(deg_sh.at[pl.ds(s * BPT, BPT)], buf)
    pltpu.sync_copy(buf, degp.at[c, pl.ds(s * BPT, BPT)])


@functools.partial(
    pl.kernel,
    out_type=jax.ShapeDtypeStruct((NC, NPAD, D), _f32),
    mesh=_mesh,
    scratch_types=[
        pltpu.VMEM((NCHUNK, CH), jnp.int32),
        pltpu.VMEM((NCHUNK, CH), jnp.int32),
        pltpu.VMEM((RCH, D), _f32),
        pltpu.VMEM_SHARED((NPAD, D), _f32),
        pltpu.SemaphoreType.DMA,
    ],
)
def _agg_call(y_hbm, src2, dst2, acc_out, sidx, didx, rows, acc_sh, sem):
    c = lax.axis_index("c")
    s = lax.axis_index("s")
    w = c * NS + s

    def _zrow(i, _):
        def _zlane(j, _):
            rows[i, pl.ds(j * 16, 16)] = jnp.zeros((16,), _f32)
            return 0
        lax.fori_loop(0, D // 16, _zlane, 0)
        return 0

    lax.fori_loop(0, RCH, _zrow, 0)
    for t in range(RPT // RCH):
        pltpu.sync_copy(rows, acc_sh.at[pl.ds(s * RPT + t * RCH, RCH)])
    plsc.subcore_barrier()

    pltpu.sync_copy(src2.at[pl.ds(w * NCHUNK, NCHUNK)], sidx)
    pltpu.sync_copy(dst2.at[pl.ds(w * NCHUNK, NCHUNK)], didx)

    def _chunk(j, _):
        pltpu.async_copy(y_hbm.at[sidx.at[j]], rows.at[pl.ds(0, CH)],
                         sem).wait()
        pltpu.sync_copy(rows.at[pl.ds(0, CH)], acc_sh.at[didx.at[j]],
                        add=True)
        return 0

    lax.fori_loop(0, NCHUNK, _chunk, 0)
    plsc.subcore_barrier()

    for t in range(RPT // RCH):
        base = s * RPT + t * RCH
        pltpu.sync_copy(acc_sh.at[pl.ds(base, RCH)], rows)
        pltpu.sync_copy(rows, acc_out.at[c, pl.ds(base, RCH)])


def _scale_body(deg_ref, x_ref, y_ref):
    d = lax.rsqrt(deg_ref[:, 0] + deg_ref[:, 1] + 1.0)
    y_ref[...] = x_ref[...] * d[:, None]


_R = 1000


def _scale_call(degT, x):
    return pl.pallas_call(
        _scale_body,
        out_shape=jax.ShapeDtypeStruct((N_NODES, D), _f32),
        grid=(N_NODES // _R,),
        in_specs=[
            pl.BlockSpec((_R, 2), lambda i: (i, 0)),
            pl.BlockSpec((_R, D), lambda i: (i, 0)),
        ],
        out_specs=pl.BlockSpec((_R, D), lambda i: (i, 0)),
    )(degT, x)


def _final_body(deg_ref, acc_ref, y_ref, w_ref, b_ref, a_ref, o_ref):
    d = lax.rsqrt(deg_ref[:, 0] + deg_ref[:, 1] + 1.0)
    sfull = (acc_ref[0] + acc_ref[1] + y_ref[...]) * d[:, None]
    z = jnp.dot(sfull, w_ref[...], preferred_element_type=_f32) + b_ref[...]
    o_ref[...] = jnp.where(z >= 0, z, a_ref[...] * z)


def _final_call(degT, acc, y, W, b2, a2):
    return pl.pallas_call(
        _final_body,
        out_shape=jax.ShapeDtypeStruct((N_NODES, D), _f32),
        grid=(N_NODES // _R,),
        in_specs=[
            pl.BlockSpec((_R, 2), lambda i: (i, 0)),
            pl.BlockSpec((NC, _R, D), lambda i: (0, i, 0)),
            pl.BlockSpec((_R, D), lambda i: (i, 0)),
            pl.BlockSpec((D, D), lambda i: (0, 0)),
            pl.BlockSpec((1, D), lambda i: (0, 0)),
            pl.BlockSpec((1, D), lambda i: (0, 0)),
        ],
        out_specs=pl.BlockSpec((_R, D), lambda i: (i, 0)),
    )(degT, acc, y, W, b2, a2)


def kernel(x, edge_index, W, b, alpha):
    ei = edge_index.astype(jnp.int32)
    src2 = ei[0].reshape(NROWCH, CH)
    dst2 = ei[1].reshape(NROWCH, CH)

    degp = _deg_call(dst2)
    degT = jnp.transpose(degp)[:N_NODES]
    y = _scale_call(degT, x)
    acc = _agg_call(y, src2, dst2)
    out = _final_call(degT, acc, y, W,
                      b.reshape(1, D), alpha.reshape(1, D))
    return out


# speedup vs baseline: 41.1145x; 1.2797x over previous
import functools

import jax
import jax.numpy as jnp
from jax import lax
from jax.experimental import pallas as pl
from jax.experimental.pallas import tpu as pltpu
from jax.experimental.pallas import tpu_sc as plsc

N_NODES = 10000
N_EDGES = 320000
D = 128

NC, NS = 2, 16
NW = NC * NS
CH = 80
EPT = N_EDGES // NW
NCHUNK = EPT // CH
NBINS = 10240
BPT = NBINS // NS
NPAD = 10240
RPT = NPAD // NS
ZCH = 80

_f32 = jnp.float32

_mesh = plsc.VectorSubcoreMesh(core_axis_name="c", subcore_axis_name="s")


@functools.partial(
    pl.kernel,
    out_type=jax.ShapeDtypeStruct((NC, NBINS), _f32),
    mesh=_mesh,
    scratch_types=[
        pltpu.VMEM((NCHUNK, CH), jnp.int32),
        pltpu.VMEM((128,), _f32),
        pltpu.VMEM((BPT,), _f32),
        pltpu.VMEM_SHARED((NBINS,), _f32),
    ],
)
def _deg_call(dst2, degp, idxv, ones_v, buf, deg_sh):
    c = lax.axis_index("c")
    s = lax.axis_index("s")
    w = c * NS + s

    for i in range(8):
        ones_v[pl.ds(i * 16, 16)] = jnp.ones((16,), _f32)

    def _z(i, _):
        buf[pl.ds(i * 16, 16)] = jnp.zeros((16,), _f32)
        return 0

    lax.fori_loop(0, BPT // 16, _z, 0)
    pltpu.sync_copy(buf, deg_sh.at[pl.ds(s * BPT, BPT)])
    plsc.subcore_barrier()

    pltpu.sync_copy(dst2.at[w], idxv)

    def _scatter(j, _):
        pltpu.sync_copy(ones_v.at[pl.ds(0, CH)], deg_sh.at[idxv.at[j]],
                        add=True)
        return 0

    lax.fori_loop(0, NCHUNK, _scatter, 0)
    plsc.subcore_barrier()

    pltpu.sync_copy(deg_sh.at[pl.ds(s * BPT, BPT)], buf)
    pltpu.sync_copy(buf, degp.at[c, pl.ds(s * BPT, BPT)])


@functools.partial(
    pl.kernel,
    out_type=jax.ShapeDtypeStruct((NC, NPAD, D), _f32),
    mesh=_mesh,
    scratch_types=[
        pltpu.VMEM((EPT,), jnp.int32),
        pltpu.VMEM((NCHUNK, CH), jnp.int32),
        pltpu.VMEM((CH, D), _f32),
        pltpu.VMEM((CH, D), _f32),
        pltpu.VMEM_SHARED((NPAD, D), _f32),
        pltpu.SemaphoreType.DMA,
        pltpu.SemaphoreType.DMA,
    ],
)
def _agg_call(y_hbm, src1, dst2, acc_out, sidx, didx, rows, rows_b, acc_sh,
              sem_a, sem_b):
    c = lax.axis_index("c")
    s = lax.axis_index("s")
    w = c * NS + s

    def _zrow(i, _):
        def _zlane(j, _):
            rows[i, pl.ds(j * 16, 16)] = jnp.zeros((16,), _f32)
            return 0
        lax.fori_loop(0, D // 16, _zlane, 0)
        return 0

    lax.fori_loop(0, CH, _zrow, 0)
    for t in range(RPT // ZCH):
        pltpu.sync_copy(rows, acc_sh.at[pl.ds(s * RPT + t * ZCH, ZCH)])
    plsc.subcore_barrier()

    pltpu.sync_copy(src1.at[pl.ds(w * EPT, EPT)], sidx)
    pltpu.sync_copy(dst2.at[w], didx)

    ra = rows
    pltpu.async_copy(y_hbm.at[sidx.at[pl.ds(0, CH)]], ra, sem_a)

    def _pair(i, _):
        pltpu.async_copy(y_hbm.at[sidx.at[pl.ds((2 * i + 1) * CH, CH)]], rows_b, sem_b)
        pltpu.make_async_copy(y_hbm.at[sidx.at[pl.ds((2 * i) * CH, CH)]], ra, sem_a).wait()
        pltpu.sync_copy(ra, acc_sh.at[didx.at[2 * i]], add=True)
        pltpu.async_copy(y_hbm.at[sidx.at[pl.ds((2 * i + 2) * CH, CH)]], ra, sem_a)
        pltpu.make_async_copy(y_hbm.at[sidx.at[pl.ds((2 * i + 1) * CH, CH)]], rows_b,
                              sem_b).wait()
        pltpu.sync_copy(rows_b, acc_sh.at[didx.at[2 * i + 1]], add=True)
        return 0

    lax.fori_loop(0, (NCHUNK - 3) // 2, _pair, 0)
    pltpu.async_copy(y_hbm.at[sidx.at[pl.ds((NCHUNK - 2) * CH, CH)]], rows_b, sem_b)
    pltpu.make_async_copy(y_hbm.at[sidx.at[pl.ds((NCHUNK - 3) * CH, CH)]], ra, sem_a).wait()
    pltpu.sync_copy(ra, acc_sh.at[didx.at[NCHUNK - 3]], add=True)
    pltpu.async_copy(y_hbm.at[sidx.at[pl.ds((NCHUNK - 1) * CH, CH)]], ra, sem_a)
    pltpu.make_async_copy(y_hbm.at[sidx.at[pl.ds((NCHUNK - 2) * CH, CH)]], rows_b, sem_b).wait()
    pltpu.sync_copy(rows_b, acc_sh.at[didx.at[NCHUNK - 2]], add=True)
    pltpu.make_async_copy(y_hbm.at[sidx.at[pl.ds((NCHUNK - 1) * CH, CH)]], ra, sem_a).wait()
    pltpu.sync_copy(ra, acc_sh.at[didx.at[NCHUNK - 1]], add=True)
    plsc.subcore_barrier()

    for t in range(RPT // ZCH):
        base = s * RPT + t * ZCH
        pltpu.sync_copy(acc_sh.at[pl.ds(base, ZCH)], rows)
        pltpu.sync_copy(rows, acc_out.at[c, pl.ds(base, ZCH)])


def _scale_body(deg_ref, x_ref, y_ref):
    d = lax.rsqrt(deg_ref[:, 0] + deg_ref[:, 1] + 1.0)
    y_ref[...] = x_ref[...] * d[:, None]


_R = 1000


def _scale_call(degT, x):
    return pl.pallas_call(
        _scale_body,
        out_shape=jax.ShapeDtypeStruct((N_NODES, D), _f32),
        grid=(N_NODES // _R,),
        in_specs=[
            pl.BlockSpec((_R, 2), lambda i: (i, 0)),
            pl.BlockSpec((_R, D), lambda i: (i, 0)),
        ],
        out_specs=pl.BlockSpec((_R, D), lambda i: (i, 0)),
    )(degT, x)


def _final_body(deg_ref, acc_ref, y_ref, w_ref, b_ref, a_ref, o_ref):
    d = lax.rsqrt(deg_ref[:, 0] + deg_ref[:, 1] + 1.0)
    sfull = (acc_ref[0] + acc_ref[1] + y_ref[...]) * d[:, None]
    z = jnp.dot(sfull, w_ref[...], preferred_element_type=_f32) + b_ref[...]
    o_ref[...] = jnp.where(z >= 0, z, a_ref[...] * z)


def _final_call(degT, acc, y, W, b2, a2):
    return pl.pallas_call(
        _final_body,
        out_shape=jax.ShapeDtypeStruct((N_NODES, D), _f32),
        grid=(N_NODES // _R,),
        in_specs=[
            pl.BlockSpec((_R, 2), lambda i: (i, 0)),
            pl.BlockSpec((NC, _R, D), lambda i: (0, i, 0)),
            pl.BlockSpec((_R, D), lambda i: (i, 0)),
            pl.BlockSpec((D, D), lambda i: (0, 0)),
            pl.BlockSpec((1, D), lambda i: (0, 0)),
            pl.BlockSpec((1, D), lambda i: (0, 0)),
        ],
        out_specs=pl.BlockSpec((_R, D), lambda i: (i, 0)),
    )(degT, acc, y, W, b2, a2)


def kernel(x, edge_index, W, b, alpha):
    ei = edge_index.astype(jnp.int32)
    src1 = ei[0]
    dst2 = ei[1].reshape(NW, NCHUNK, CH)

    degp = _deg_call(dst2)
    degT = jnp.transpose(degp)[:N_NODES]
    y = _scale_call(degT, x)
    acc = _agg_call(y, src1, dst2)
    out = _final_call(degT, acc, y, W,
                      b.reshape(1, D), alpha.reshape(1, D))
    return out


# speedup vs baseline: 41.6146x; 1.0122x over previous
import functools

import jax
import jax.numpy as jnp
from jax import lax
from jax.experimental import pallas as pl
from jax.experimental.pallas import tpu as pltpu
from jax.experimental.pallas import tpu_sc as plsc

N_NODES = 10000
N_EDGES = 320000
D = 128

NC, NS = 2, 16
NW = NC * NS
CH = 80
EPT = N_EDGES // NW
NCHUNK = EPT // CH
NBINS = 10240
BPT = NBINS // NS
NPAD = 10240
RPT = NPAD // NS
ZCH = 80

_f32 = jnp.float32

_mesh = plsc.VectorSubcoreMesh(core_axis_name="c", subcore_axis_name="s")


@functools.partial(
    pl.kernel,
    out_type=jax.ShapeDtypeStruct((NC, NBINS), _f32),
    mesh=_mesh,
    scratch_types=[
        pltpu.VMEM((NCHUNK, CH), jnp.int32),
        pltpu.VMEM((128,), _f32),
        pltpu.VMEM((BPT,), _f32),
        pltpu.VMEM_SHARED((NBINS,), _f32),
    ],
)
def _deg_call(dst2, degp, idxv, ones_v, buf, deg_sh):
    c = lax.axis_index("c")
    s = lax.axis_index("s")
    w = c * NS + s

    for i in range(8):
        ones_v[pl.ds(i * 16, 16)] = jnp.ones((16,), _f32)

    def _z(i, _):
        buf[pl.ds(i * 16, 16)] = jnp.zeros((16,), _f32)
        return 0

    lax.fori_loop(0, BPT // 16, _z, 0)
    pltpu.sync_copy(buf, deg_sh.at[pl.ds(s * BPT, BPT)])
    plsc.subcore_barrier()

    pltpu.sync_copy(dst2.at[w], idxv)

    def _scatter(j, _):
        pltpu.sync_copy(ones_v.at[pl.ds(0, CH)], deg_sh.at[idxv.at[j]],
                        add=True)
        return 0

    lax.fori_loop(0, NCHUNK, _scatter, 0)
    plsc.subcore_barrier()

    pltpu.sync_copy(deg_sh.at[pl.ds(s * BPT, BPT)], buf)
    pltpu.sync_copy(buf, degp.at[c, pl.ds(s * BPT, BPT)])


@functools.partial(
    pl.kernel,
    out_type=jax.ShapeDtypeStruct((NC, NPAD, D), _f32),
    mesh=_mesh,
    scratch_types=[
        pltpu.VMEM((EPT,), jnp.int32),
        pltpu.VMEM((NCHUNK, CH), jnp.int32),
        pltpu.VMEM((CH, D), _f32),
        pltpu.VMEM((CH, D), _f32),
        pltpu.VMEM_SHARED((NPAD, D), _f32),
        pltpu.SemaphoreType.DMA,
        pltpu.SemaphoreType.DMA,
    ],
)
def _agg_call(y_hbm, src1, dst2, acc_out, sidx, didx, rows, rows_b, acc_sh,
              sem_a, sem_b):
    c = lax.axis_index("c")
    s = lax.axis_index("s")
    w = c * NS + s

    def _zrow(i, _):
        def _zlane(j, _):
            rows[i, pl.ds(j * 16, 16)] = jnp.zeros((16,), _f32)
            return 0
        lax.fori_loop(0, D // 16, _zlane, 0)
        return 0

    lax.fori_loop(0, CH, _zrow, 0)
    for t in range(RPT // ZCH):
        pltpu.async_copy(rows, acc_sh.at[pl.ds(s * RPT + t * ZCH, ZCH)],
                         sem_a)
    for t in range(RPT // ZCH):
        pltpu.make_async_copy(rows, acc_sh.at[pl.ds(s * RPT + t * ZCH, ZCH)],
                              sem_a).wait()
    plsc.subcore_barrier()

    pltpu.sync_copy(src1.at[pl.ds(w * EPT, EPT)], sidx)
    pltpu.sync_copy(dst2.at[w], didx)

    ra = rows
    pltpu.async_copy(y_hbm.at[sidx.at[pl.ds(0, CH)]], ra, sem_a)

    def _pair(i, _):
        pltpu.async_copy(y_hbm.at[sidx.at[pl.ds((2 * i + 1) * CH, CH)]], rows_b, sem_b)
        pltpu.make_async_copy(y_hbm.at[sidx.at[pl.ds((2 * i) * CH, CH)]], ra, sem_a).wait()
        pltpu.sync_copy(ra, acc_sh.at[didx.at[2 * i]], add=True)
        pltpu.async_copy(y_hbm.at[sidx.at[pl.ds((2 * i + 2) * CH, CH)]], ra, sem_a)
        pltpu.make_async_copy(y_hbm.at[sidx.at[pl.ds((2 * i + 1) * CH, CH)]], rows_b,
                              sem_b).wait()
        pltpu.sync_copy(rows_b, acc_sh.at[didx.at[2 * i + 1]], add=True)
        return 0

    lax.fori_loop(0, (NCHUNK - 3) // 2, _pair, 0)
    pltpu.async_copy(y_hbm.at[sidx.at[pl.ds((NCHUNK - 2) * CH, CH)]], rows_b, sem_b)
    pltpu.make_async_copy(y_hbm.at[sidx.at[pl.ds((NCHUNK - 3) * CH, CH)]], ra, sem_a).wait()
    pltpu.sync_copy(ra, acc_sh.at[didx.at[NCHUNK - 3]], add=True)
    pltpu.async_copy(y_hbm.at[sidx.at[pl.ds((NCHUNK - 1) * CH, CH)]], ra, sem_a)
    pltpu.make_async_copy(y_hbm.at[sidx.at[pl.ds((NCHUNK - 2) * CH, CH)]], rows_b, sem_b).wait()
    pltpu.sync_copy(rows_b, acc_sh.at[didx.at[NCHUNK - 2]], add=True)
    pltpu.make_async_copy(y_hbm.at[sidx.at[pl.ds((NCHUNK - 1) * CH, CH)]], ra, sem_a).wait()
    pltpu.sync_copy(ra, acc_sh.at[didx.at[NCHUNK - 1]], add=True)
    plsc.subcore_barrier()

    bufs = (rows, rows_b)
    nt = RPT // ZCH
    for t in range(nt):
        buf = bufs[t & 1]
        if t >= 2:
            pltpu.make_async_copy(
                buf, acc_out.at[c, pl.ds(s * RPT + (t - 2) * ZCH, ZCH)],
                sem_a).wait()
        pltpu.sync_copy(acc_sh.at[pl.ds(s * RPT + t * ZCH, ZCH)], buf)
        pltpu.async_copy(buf, acc_out.at[c, pl.ds(s * RPT + t * ZCH, ZCH)],
                         sem_a)
    for t in (nt - 2, nt - 1):
        pltpu.make_async_copy(
            bufs[t & 1], acc_out.at[c, pl.ds(s * RPT + t * ZCH, ZCH)],
            sem_a).wait()


def _scale_body(deg_ref, x_ref, y_ref):
    d = lax.rsqrt(deg_ref[:, 0] + deg_ref[:, 1] + 1.0)
    y_ref[...] = x_ref[...] * d[:, None]


_R = 1000


def _scale_call(degT, x):
    return pl.pallas_call(
        _scale_body,
        out_shape=jax.ShapeDtypeStruct((N_NODES, D), _f32),
        grid=(N_NODES // _R,),
        in_specs=[
            pl.BlockSpec((_R, 2), lambda i: (i, 0)),
            pl.BlockSpec((_R, D), lambda i: (i, 0)),
        ],
        out_specs=pl.BlockSpec((_R, D), lambda i: (i, 0)),
    )(degT, x)


def _final_body(deg_ref, acc_ref, y_ref, w_ref, b_ref, a_ref, o_ref):
    d = lax.rsqrt(deg_ref[:, 0] + deg_ref[:, 1] + 1.0)
    sfull = (acc_ref[0] + acc_ref[1] + y_ref[...]) * d[:, None]
    z = jnp.dot(sfull, w_ref[...], preferred_element_type=_f32) + b_ref[...]
    o_ref[...] = jnp.where(z >= 0, z, a_ref[...] * z)


def _final_call(degT, acc, y, W, b2, a2):
    return pl.pallas_call(
        _final_body,
        out_shape=jax.ShapeDtypeStruct((N_NODES, D), _f32),
        grid=(N_NODES // _R,),
        in_specs=[
            pl.BlockSpec((_R, 2), lambda i: (i, 0)),
            pl.BlockSpec((NC, _R, D), lambda i: (0, i, 0)),
            pl.BlockSpec((_R, D), lambda i: (i, 0)),
            pl.BlockSpec((D, D), lambda i: (0, 0)),
            pl.BlockSpec((1, D), lambda i: (0, 0)),
            pl.BlockSpec((1, D), lambda i: (0, 0)),
        ],
        out_specs=pl.BlockSpec((_R, D), lambda i: (i, 0)),
    )(degT, acc, y, W, b2, a2)


def kernel(x, edge_index, W, b, alpha):
    ei = edge_index.astype(jnp.int32)
    src1 = ei[0]
    dst2 = ei[1].reshape(NW, NCHUNK, CH)

    degp = _deg_call(dst2)
    degT = jnp.transpose(degp)[:N_NODES]
    y = _scale_call(degT, x)
    acc = _agg_call(y, src1, dst2)
    out = _final_call(degT, acc, y, W,
                      b.reshape(1, D), alpha.reshape(1, D))
    return out


# speedup vs baseline: 43.8181x; 1.0530x over previous
import functools

import jax
import jax.numpy as jnp
from jax import lax
from jax.experimental import pallas as pl
from jax.experimental.pallas import tpu as pltpu
from jax.experimental.pallas import tpu_sc as plsc

N_NODES = 10000
N_EDGES = 320000
D = 128

NC, NS = 2, 16
NW = NC * NS
CH = 80
EPT = N_EDGES // NW
NCHUNK = EPT // CH
NBINS = 10240
BPT = NBINS // NS
NPAD = 10240
RPT = NPAD // NS
ZCH = 80

_f32 = jnp.float32

_mesh = plsc.VectorSubcoreMesh(core_axis_name="c", subcore_axis_name="s")


@functools.partial(
    pl.kernel,
    out_type=jax.ShapeDtypeStruct((NC, NBINS), _f32),
    mesh=_mesh,
    scratch_types=[
        pltpu.VMEM((EPT,), jnp.int32),
        pltpu.VMEM((128,), _f32),
        pltpu.VMEM((BPT,), _f32),
        pltpu.VMEM_SHARED((NBINS,), _f32),
    ],
)
def _deg_call(dst1, degp, idxv, ones_v, buf, deg_sh):
    c = lax.axis_index("c")
    s = lax.axis_index("s")
    w = c * NS + s

    for i in range(8):
        ones_v[pl.ds(i * 16, 16)] = jnp.ones((16,), _f32)

    def _z(i, _):
        buf[pl.ds(i * 16, 16)] = jnp.zeros((16,), _f32)
        return 0

    lax.fori_loop(0, BPT // 16, _z, 0)
    pltpu.sync_copy(buf, deg_sh.at[pl.ds(s * BPT, BPT)])
    plsc.subcore_barrier()

    pltpu.sync_copy(dst1.at[pl.ds(w * EPT, EPT)], idxv)

    def _scatter(j, _):
        pltpu.sync_copy(ones_v.at[pl.ds(0, CH)],
                        deg_sh.at[idxv.at[pl.ds(j * CH, CH)]], add=True)
        return 0

    lax.fori_loop(0, NCHUNK, _scatter, 0)
    plsc.subcore_barrier()

    pltpu.sync_copy(deg_sh.at[pl.ds(s * BPT, BPT)], buf)
    pltpu.sync_copy(buf, degp.at[c, pl.ds(s * BPT, BPT)])


@functools.partial(
    pl.kernel,
    out_type=jax.ShapeDtypeStruct((NC, NPAD, D), _f32),
    mesh=_mesh,
    scratch_types=[
        pltpu.VMEM((EPT,), jnp.int32),
        pltpu.VMEM((EPT,), jnp.int32),
        pltpu.VMEM((CH, D), _f32),
        pltpu.VMEM((CH, D), _f32),
        pltpu.VMEM_SHARED((NPAD, D), _f32),
        pltpu.SemaphoreType.DMA,
        pltpu.SemaphoreType.DMA,
    ],
)
def _agg_call(y_hbm, src1, dst1, acc_out, sidx, didx, rows, rows_b, acc_sh,
              sem_a, sem_b):
    c = lax.axis_index("c")
    s = lax.axis_index("s")
    w = c * NS + s

    def _zrow(i, _):
        def _zlane(j, _):
            rows[i, pl.ds(j * 16, 16)] = jnp.zeros((16,), _f32)
            return 0
        lax.fori_loop(0, D // 16, _zlane, 0)
        return 0

    lax.fori_loop(0, CH, _zrow, 0)
    for t in range(RPT // ZCH):
        pltpu.async_copy(rows, acc_sh.at[pl.ds(s * RPT + t * ZCH, ZCH)],
                         sem_a)
    for t in range(RPT // ZCH):
        pltpu.make_async_copy(rows, acc_sh.at[pl.ds(s * RPT + t * ZCH, ZCH)],
                              sem_a).wait()
    plsc.subcore_barrier()

    pltpu.sync_copy(src1.at[pl.ds(w * EPT, EPT)], sidx)
    pltpu.sync_copy(dst1.at[pl.ds(w * EPT, EPT)], didx)

    ra = rows
    pltpu.async_copy(y_hbm.at[sidx.at[pl.ds(0, CH)]], ra, sem_a)

    def _pair(i, _):
        pltpu.async_copy(y_hbm.at[sidx.at[pl.ds((2 * i + 1) * CH, CH)]], rows_b, sem_b)
        pltpu.make_async_copy(y_hbm.at[sidx.at[pl.ds((2 * i) * CH, CH)]], ra, sem_a).wait()
        pltpu.sync_copy(ra, acc_sh.at[didx.at[pl.ds((2 * i) * CH, CH)]], add=True)
        pltpu.async_copy(y_hbm.at[sidx.at[pl.ds((2 * i + 2) * CH, CH)]], ra, sem_a)
        pltpu.make_async_copy(y_hbm.at[sidx.at[pl.ds((2 * i + 1) * CH, CH)]], rows_b,
                              sem_b).wait()
        pltpu.sync_copy(rows_b, acc_sh.at[didx.at[pl.ds((2 * i + 1) * CH, CH)]], add=True)
        return 0

    lax.fori_loop(0, (NCHUNK - 3) // 2, _pair, 0)
    pltpu.async_copy(y_hbm.at[sidx.at[pl.ds((NCHUNK - 2) * CH, CH)]], rows_b, sem_b)
    pltpu.make_async_copy(y_hbm.at[sidx.at[pl.ds((NCHUNK - 3) * CH, CH)]], ra, sem_a).wait()
    pltpu.sync_copy(ra, acc_sh.at[didx.at[pl.ds((NCHUNK - 3) * CH, CH)]], add=True)
    pltpu.async_copy(y_hbm.at[sidx.at[pl.ds((NCHUNK - 1) * CH, CH)]], ra, sem_a)
    pltpu.make_async_copy(y_hbm.at[sidx.at[pl.ds((NCHUNK - 2) * CH, CH)]], rows_b, sem_b).wait()
    pltpu.sync_copy(rows_b, acc_sh.at[didx.at[pl.ds((NCHUNK - 2) * CH, CH)]], add=True)
    pltpu.make_async_copy(y_hbm.at[sidx.at[pl.ds((NCHUNK - 1) * CH, CH)]], ra, sem_a).wait()
    pltpu.sync_copy(ra, acc_sh.at[didx.at[pl.ds((NCHUNK - 1) * CH, CH)]], add=True)
    plsc.subcore_barrier()

    bufs = (rows, rows_b)
    nt = RPT // ZCH
    for t in range(nt):
        buf = bufs[t & 1]
        if t >= 2:
            pltpu.make_async_copy(
                buf, acc_out.at[c, pl.ds(s * RPT + (t - 2) * ZCH, ZCH)],
                sem_a).wait()
        pltpu.sync_copy(acc_sh.at[pl.ds(s * RPT + t * ZCH, ZCH)], buf)
        pltpu.async_copy(buf, acc_out.at[c, pl.ds(s * RPT + t * ZCH, ZCH)],
                         sem_a)
    for t in (nt - 2, nt - 1):
        pltpu.make_async_copy(
            bufs[t & 1], acc_out.at[c, pl.ds(s * RPT + t * ZCH, ZCH)],
            sem_a).wait()


def _scale_body(deg_ref, x_ref, y_ref, d_ref):
    dsum = deg_ref[0, :] + deg_ref[1, :] + 1.0
    dlane = lax.rsqrt(dsum).reshape(1, NBINS)
    dsub = jnp.transpose(dlane)[:N_NODES]
    y_ref[...] = x_ref[...] * dsub
    d_ref[...] = jnp.broadcast_to(dsub, (N_NODES, 8))


_R = 1000


def _scale_call(degp, x):
    return pl.pallas_call(
        _scale_body,
        out_shape=[
            jax.ShapeDtypeStruct((N_NODES, D), _f32),
            jax.ShapeDtypeStruct((N_NODES, 8), _f32),
        ],
        grid=(1,),
        in_specs=[
            pl.BlockSpec((NC, NBINS), lambda i: (0, 0)),
            pl.BlockSpec((N_NODES, D), lambda i: (0, 0)),
        ],
        out_specs=[
            pl.BlockSpec((N_NODES, D), lambda i: (0, 0)),
            pl.BlockSpec((N_NODES, 8), lambda i: (0, 0)),
        ],
    )(degp, x)


def _final_body(d_ref, acc_ref, y_ref, w_ref, b_ref, a_ref, o_ref):
    d = d_ref[:, 0:1]
    sfull = (acc_ref[0] + acc_ref[1] + y_ref[...]) * d
    z = jnp.dot(sfull, w_ref[...], preferred_element_type=_f32) + b_ref[...]
    o_ref[...] = jnp.where(z >= 0, z, a_ref[...] * z)


def _final_call(dcol, acc, y, W, b2, a2):
    return pl.pallas_call(
        _final_body,
        out_shape=jax.ShapeDtypeStruct((N_NODES, D), _f32),
        grid=(N_NODES // _R,),
        in_specs=[
            pl.BlockSpec((_R, 8), lambda i: (i, 0)),
            pl.BlockSpec((NC, _R, D), lambda i: (0, i, 0)),
            pl.BlockSpec((_R, D), lambda i: (i, 0)),
            pl.BlockSpec((D, D), lambda i: (0, 0)),
            pl.BlockSpec((1, D), lambda i: (0, 0)),
            pl.BlockSpec((1, D), lambda i: (0, 0)),
        ],
        out_specs=pl.BlockSpec((_R, D), lambda i: (i, 0)),
    )(dcol, acc, y, W, b2, a2)


def kernel(x, edge_index, W, b, alpha):
    ei = edge_index.astype(jnp.int32)
    src1 = ei[0]
    dst1 = ei[1]

    degp = _deg_call(dst1)
    y, dcol = _scale_call(degp, x)
    acc = _agg_call(y, src1, dst1)
    out = _final_call(dcol, acc, y, W,
                      b.reshape(1, D), alpha.reshape(1, D))
    return out


# speedup vs baseline: 47.0854x; 1.0746x over previous
import functools

import jax
import jax.numpy as jnp
from jax import lax
from jax.experimental import pallas as pl
from jax.experimental.pallas import tpu as pltpu
from jax.experimental.pallas import tpu_sc as plsc

N_NODES = 10000
N_EDGES = 320000
D = 128

NC, NS = 2, 16
NW = NC * NS
CH = 80
EPT = N_EDGES // NW
NCHUNK = EPT // CH
NBINS = 10240
BPT = NBINS // NS
NPAD = 10240
RPT = NPAD // NS
ZCH = 80

_f32 = jnp.float32

_mesh = plsc.VectorSubcoreMesh(core_axis_name="c", subcore_axis_name="s")


@functools.partial(
    pl.kernel,
    out_type=jax.ShapeDtypeStruct((NC, NBINS), _f32),
    mesh=_mesh,
    scratch_types=[
        pltpu.VMEM((EPT,), jnp.int32),
        pltpu.VMEM((128,), _f32),
        pltpu.VMEM((BPT,), _f32),
        pltpu.VMEM_SHARED((NBINS,), _f32),
    ],
)
def _deg_call(dst1, degp, idxv, ones_v, buf, deg_sh):
    c = lax.axis_index("c")
    s = lax.axis_index("s")
    w = c * NS + s

    for i in range(8):
        ones_v[pl.ds(i * 16, 16)] = jnp.ones((16,), _f32)

    def _z(i, _):
        buf[pl.ds(i * 16, 16)] = jnp.zeros((16,), _f32)
        return 0

    lax.fori_loop(0, BPT // 16, _z, 0)
    pltpu.sync_copy(buf, deg_sh.at[pl.ds(s * BPT, BPT)])
    plsc.subcore_barrier()

    pltpu.sync_copy(dst1.at[pl.ds(w * EPT, EPT)], idxv)

    def _scatter(j, _):
        pltpu.sync_copy(ones_v.at[pl.ds(0, CH)],
                        deg_sh.at[idxv.at[pl.ds(j * CH, CH)]], add=True)
        return 0

    lax.fori_loop(0, NCHUNK, _scatter, 0)
    plsc.subcore_barrier()

    pltpu.sync_copy(deg_sh.at[pl.ds(s * BPT, BPT)], buf)
    pltpu.sync_copy(buf, degp.at[c, pl.ds(s * BPT, BPT)])


@functools.partial(
    pl.kernel,
    out_type=jax.ShapeDtypeStruct((NC, NPAD, D), _f32),
    mesh=_mesh,
    scratch_types=[
        pltpu.VMEM((EPT,), jnp.int32),
        pltpu.VMEM((EPT,), jnp.int32),
        pltpu.VMEM((CH, D), _f32),
        pltpu.VMEM((CH, D), _f32),
        pltpu.VMEM_SHARED((NPAD, D), _f32),
        pltpu.SemaphoreType.DMA,
        pltpu.SemaphoreType.DMA,
    ],
)
def _agg_call(y_hbm, src1, dst1, acc_out, sidx, didx, rows, rows_b, acc_sh,
              sem_a, sem_b):
    c = lax.axis_index("c")
    s = lax.axis_index("s")
    w = c * NS + s

    def _zrow(i, _):
        def _zlane(j, _):
            rows[i, pl.ds(j * 16, 16)] = jnp.zeros((16,), _f32)
            return 0
        lax.fori_loop(0, D // 16, _zlane, 0)
        return 0

    lax.fori_loop(0, CH, _zrow, 0)
    for t in range(RPT // ZCH):
        pltpu.async_copy(rows, acc_sh.at[pl.ds(s * RPT + t * ZCH, ZCH)],
                         sem_a)
    for t in range(RPT // ZCH):
        pltpu.make_async_copy(rows, acc_sh.at[pl.ds(s * RPT + t * ZCH, ZCH)],
                              sem_a).wait()
    plsc.subcore_barrier()

    pltpu.sync_copy(src1.at[pl.ds(w * EPT, EPT)], sidx)
    pltpu.sync_copy(dst1.at[pl.ds(w * EPT, EPT)], didx)

    ra = rows
    pltpu.async_copy(y_hbm.at[sidx.at[pl.ds(0, CH)]], ra, sem_a)

    def _pair(i, _):
        pltpu.async_copy(y_hbm.at[sidx.at[pl.ds((2 * i + 1) * CH, CH)]], rows_b, sem_b)
        pltpu.make_async_copy(y_hbm.at[sidx.at[pl.ds((2 * i) * CH, CH)]], ra, sem_a).wait()
        pltpu.sync_copy(ra, acc_sh.at[didx.at[pl.ds((2 * i) * CH, CH)]], add=True)
        pltpu.async_copy(y_hbm.at[sidx.at[pl.ds((2 * i + 2) * CH, CH)]], ra, sem_a)
        pltpu.make_async_copy(y_hbm.at[sidx.at[pl.ds((2 * i + 1) * CH, CH)]], rows_b,
                              sem_b).wait()
        pltpu.sync_copy(rows_b, acc_sh.at[didx.at[pl.ds((2 * i + 1) * CH, CH)]], add=True)
        return 0

    lax.fori_loop(0, (NCHUNK - 3) // 2, _pair, 0)
    pltpu.async_copy(y_hbm.at[sidx.at[pl.ds((NCHUNK - 2) * CH, CH)]], rows_b, sem_b)
    pltpu.make_async_copy(y_hbm.at[sidx.at[pl.ds((NCHUNK - 3) * CH, CH)]], ra, sem_a).wait()
    pltpu.sync_copy(ra, acc_sh.at[didx.at[pl.ds((NCHUNK - 3) * CH, CH)]], add=True)
    pltpu.async_copy(y_hbm.at[sidx.at[pl.ds((NCHUNK - 1) * CH, CH)]], ra, sem_a)
    pltpu.make_async_copy(y_hbm.at[sidx.at[pl.ds((NCHUNK - 2) * CH, CH)]], rows_b, sem_b).wait()
    pltpu.sync_copy(rows_b, acc_sh.at[didx.at[pl.ds((NCHUNK - 2) * CH, CH)]], add=True)
    pltpu.make_async_copy(y_hbm.at[sidx.at[pl.ds((NCHUNK - 1) * CH, CH)]], ra, sem_a).wait()
    pltpu.sync_copy(ra, acc_sh.at[didx.at[pl.ds((NCHUNK - 1) * CH, CH)]], add=True)
    plsc.subcore_barrier()

    bufs = (rows, rows_b)
    nt = RPT // ZCH
    for t in range(nt):
        buf = bufs[t & 1]
        if t >= 2:
            pltpu.make_async_copy(
                buf, acc_out.at[c, pl.ds(s * RPT + (t - 2) * ZCH, ZCH)],
                sem_a).wait()
        pltpu.sync_copy(acc_sh.at[pl.ds(s * RPT + t * ZCH, ZCH)], buf)
        pltpu.async_copy(buf, acc_out.at[c, pl.ds(s * RPT + t * ZCH, ZCH)],
                         sem_a)
    for t in (nt - 2, nt - 1):
        pltpu.make_async_copy(
            bufs[t & 1], acc_out.at[c, pl.ds(s * RPT + t * ZCH, ZCH)],
            sem_a).wait()


def _split_body(ei_ref, src_ref, dst_ref):
    src_ref[...] = ei_ref[0, :]
    dst_ref[...] = ei_ref[1, :]


def _split_call(ei):
    return pl.pallas_call(
        _split_body,
        out_shape=[
            jax.ShapeDtypeStruct((N_EDGES,), jnp.int32),
            jax.ShapeDtypeStruct((N_EDGES,), jnp.int32),
        ],
    )(ei)


def _scale_body(deg_ref, x_ref, y_ref, d_ref):
    dsum = deg_ref[0, :] + deg_ref[1, :] + 1.0
    dlane = lax.rsqrt(dsum).reshape(1, NBINS)
    dsub = jnp.transpose(dlane)[:N_NODES]
    y_ref[...] = x_ref[...] * dsub
    d_ref[...] = jnp.broadcast_to(dsub, (N_NODES, 8))


_R = 1000


def _scale_call(degp, x):
    return pl.pallas_call(
        _scale_body,
        out_shape=[
            jax.ShapeDtypeStruct((N_NODES, D), _f32),
            jax.ShapeDtypeStruct((N_NODES, 8), _f32),
        ],
        grid=(1,),
        in_specs=[
            pl.BlockSpec((NC, NBINS), lambda i: (0, 0)),
            pl.BlockSpec((N_NODES, D), lambda i: (0, 0)),
        ],
        out_specs=[
            pl.BlockSpec((N_NODES, D), lambda i: (0, 0)),
            pl.BlockSpec((N_NODES, 8), lambda i: (0, 0)),
        ],
    )(degp, x)


def _final_body(d_ref, acc_ref, y_ref, w_ref, b_ref, a_ref, o_ref):
    d = d_ref[:, 0:1]
    sfull = (acc_ref[0] + acc_ref[1] + y_ref[...]) * d
    z = jnp.dot(sfull, w_ref[...], preferred_element_type=_f32) + b_ref[...]
    o_ref[...] = jnp.where(z >= 0, z, a_ref[...] * z)


def _final_call(dcol, acc, y, W, b2, a2):
    return pl.pallas_call(
        _final_body,
        out_shape=jax.ShapeDtypeStruct((N_NODES, D), _f32),
        grid=(N_NODES // _R,),
        in_specs=[
            pl.BlockSpec((_R, 8), lambda i: (i, 0)),
            pl.BlockSpec((NC, _R, D), lambda i: (0, i, 0)),
            pl.BlockSpec((_R, D), lambda i: (i, 0)),
            pl.BlockSpec((D, D), lambda i: (0, 0)),
            pl.BlockSpec((1, D), lambda i: (0, 0)),
            pl.BlockSpec((1, D), lambda i: (0, 0)),
        ],
        out_specs=pl.BlockSpec((_R, D), lambda i: (i, 0)),
    )(dcol, acc, y, W, b2, a2)


def kernel(x, edge_index, W, b, alpha):
    ei = edge_index.astype(jnp.int32)
    src1, dst1 = _split_call(ei)

    degp = _deg_call(dst1)
    y, dcol = _scale_call(degp, x)
    acc = _agg_call(y, src1, dst1)
    out = _final_call(dcol, acc, y, W,
                      b.reshape(1, D), alpha.reshape(1, D))
    return out


# speedup vs baseline: 47.8045x; 1.0153x over previous
import functools

import jax
import jax.numpy as jnp
from jax import lax
from jax.experimental import pallas as pl
from jax.experimental.pallas import tpu as pltpu
from jax.experimental.pallas import tpu_sc as plsc

N_NODES = 10000
N_EDGES = 320000
D = 128

NC, NS = 2, 16
NW = NC * NS
CH = 80
EPT = N_EDGES // NW
NCHUNK = EPT // CH
NBINS = 10240
BPT = NBINS // NS
NPAD = 10240
RPT = NPAD // NS
ZCH = 80

_f32 = jnp.float32

_mesh = plsc.VectorSubcoreMesh(core_axis_name="c", subcore_axis_name="s")


@functools.partial(
    pl.kernel,
    out_type=jax.ShapeDtypeStruct((NC, NBINS), _f32),
    mesh=_mesh,
    scratch_types=[
        pltpu.VMEM((EPT,), jnp.int32),
        pltpu.VMEM((128,), _f32),
        pltpu.VMEM((BPT,), _f32),
        pltpu.VMEM_SHARED((NBINS,), _f32),
    ],
)
def _deg_call(dst1, degp, idxv, ones_v, buf, deg_sh):
    c = lax.axis_index("c")
    s = lax.axis_index("s")
    w = c * NS + s

    for i in range(8):
        ones_v[pl.ds(i * 16, 16)] = jnp.ones((16,), _f32)

    def _z(i, _):
        buf[pl.ds(i * 16, 16)] = jnp.zeros((16,), _f32)
        return 0

    lax.fori_loop(0, BPT // 16, _z, 0)
    pltpu.sync_copy(buf, deg_sh.at[pl.ds(s * BPT, BPT)])
    plsc.subcore_barrier()

    pltpu.sync_copy(dst1.at[pl.ds(w * EPT, EPT)], idxv)

    def _scatter(j, _):
        pltpu.sync_copy(ones_v.at[pl.ds(0, CH)],
                        deg_sh.at[idxv.at[pl.ds(j * CH, CH)]], add=True)
        return 0

    lax.fori_loop(0, NCHUNK, _scatter, 0)
    plsc.subcore_barrier()

    pltpu.sync_copy(deg_sh.at[pl.ds(s * BPT, BPT)], buf)
    pltpu.sync_copy(buf, degp.at[c, pl.ds(s * BPT, BPT)])


@functools.partial(
    pl.kernel,
    out_type=jax.ShapeDtypeStruct((NC, NPAD, D), _f32),
    mesh=_mesh,
    scratch_types=[
        pltpu.VMEM((EPT,), jnp.int32),
        pltpu.VMEM((EPT,), jnp.int32),
        pltpu.VMEM((CH, D), _f32),
        pltpu.VMEM((CH, D), _f32),
        pltpu.VMEM_SHARED((NPAD, D), _f32),
        pltpu.SemaphoreType.DMA,
        pltpu.SemaphoreType.DMA,
    ],
)
def _agg_call(y_hbm, src1, dst1, acc_out, sidx, didx, rows, rows_b, acc_sh,
              sem_a, sem_b):
    c = lax.axis_index("c")
    s = lax.axis_index("s")
    w = c * NS + s

    def _zrow(i, _):
        def _zlane(j, _):
            rows[i, pl.ds(j * 16, 16)] = jnp.zeros((16,), _f32)
            return 0
        lax.fori_loop(0, D // 16, _zlane, 0)
        return 0

    lax.fori_loop(0, CH, _zrow, 0)
    for t in range(RPT // ZCH):
        pltpu.async_copy(rows, acc_sh.at[pl.ds(s * RPT + t * ZCH, ZCH)],
                         sem_a)
    pltpu.async_copy(src1.at[pl.ds(w * EPT, EPT)], sidx, sem_b)
    pltpu.sync_copy(dst1.at[pl.ds(w * EPT, EPT)], didx)
    pltpu.make_async_copy(src1.at[pl.ds(w * EPT, EPT)], sidx, sem_b).wait()
    for t in range(RPT // ZCH):
        pltpu.make_async_copy(rows, acc_sh.at[pl.ds(s * RPT + t * ZCH, ZCH)],
                              sem_a).wait()
    plsc.subcore_barrier()

    ra = rows
    pltpu.async_copy(y_hbm.at[sidx.at[pl.ds(0, CH)]], ra, sem_a)

    def _pair(i, _):
        pltpu.async_copy(y_hbm.at[sidx.at[pl.ds((2 * i + 1) * CH, CH)]], rows_b, sem_b)
        pltpu.make_async_copy(y_hbm.at[sidx.at[pl.ds((2 * i) * CH, CH)]], ra, sem_a).wait()
        pltpu.sync_copy(ra, acc_sh.at[didx.at[pl.ds((2 * i) * CH, CH)]], add=True)
        pltpu.async_copy(y_hbm.at[sidx.at[pl.ds((2 * i + 2) * CH, CH)]], ra, sem_a)
        pltpu.make_async_copy(y_hbm.at[sidx.at[pl.ds((2 * i + 1) * CH, CH)]], rows_b,
                              sem_b).wait()
        pltpu.sync_copy(rows_b, acc_sh.at[didx.at[pl.ds((2 * i + 1) * CH, CH)]], add=True)
        return 0

    lax.fori_loop(0, (NCHUNK - 3) // 2, _pair, 0)
    pltpu.async_copy(y_hbm.at[sidx.at[pl.ds((NCHUNK - 2) * CH, CH)]], rows_b, sem_b)
    pltpu.make_async_copy(y_hbm.at[sidx.at[pl.ds((NCHUNK - 3) * CH, CH)]], ra, sem_a).wait()
    pltpu.sync_copy(ra, acc_sh.at[didx.at[pl.ds((NCHUNK - 3) * CH, CH)]], add=True)
    pltpu.async_copy(y_hbm.at[sidx.at[pl.ds((NCHUNK - 1) * CH, CH)]], ra, sem_a)
    pltpu.make_async_copy(y_hbm.at[sidx.at[pl.ds((NCHUNK - 2) * CH, CH)]], rows_b, sem_b).wait()
    pltpu.sync_copy(rows_b, acc_sh.at[didx.at[pl.ds((NCHUNK - 2) * CH, CH)]], add=True)
    pltpu.make_async_copy(y_hbm.at[sidx.at[pl.ds((NCHUNK - 1) * CH, CH)]], ra, sem_a).wait()
    pltpu.sync_copy(ra, acc_sh.at[didx.at[pl.ds((NCHUNK - 1) * CH, CH)]], add=True)
    plsc.subcore_barrier()

    bufs = (rows, rows_b)
    nt = RPT // ZCH
    pltpu.async_copy(acc_sh.at[pl.ds(s * RPT, ZCH)], bufs[0], sem_b)
    for t in range(nt):
        buf = bufs[t & 1]
        if t >= 2:
            pltpu.make_async_copy(
                buf, acc_out.at[c, pl.ds(s * RPT + (t - 2) * ZCH, ZCH)],
                sem_a).wait()
        if t + 1 < nt:
            pltpu.async_copy(
                acc_sh.at[pl.ds(s * RPT + (t + 1) * ZCH, ZCH)],
                bufs[(t + 1) & 1], sem_b)
        pltpu.make_async_copy(acc_sh.at[pl.ds(s * RPT + t * ZCH, ZCH)], buf,
                              sem_b).wait()
        pltpu.async_copy(buf, acc_out.at[c, pl.ds(s * RPT + t * ZCH, ZCH)],
                         sem_a)
    for t in (nt - 2, nt - 1):
        pltpu.make_async_copy(
            bufs[t & 1], acc_out.at[c, pl.ds(s * RPT + t * ZCH, ZCH)],
            sem_a).wait()


def _split_row(r):
    def body(ei_ref, o_ref):
        o_ref[...] = ei_ref[r, :]
    return body


def _split_call(ei, r):
    return pl.pallas_call(
        _split_row(r),
        out_shape=jax.ShapeDtypeStruct((N_EDGES,), jnp.int32),
    )(ei)


def _scale_body(deg_ref, x_ref, y_ref, d_ref):
    dsum = deg_ref[0, :] + deg_ref[1, :] + 1.0
    dlane = lax.rsqrt(dsum).reshape(1, NBINS)
    dsub = jnp.transpose(dlane)[:N_NODES]
    y_ref[...] = x_ref[...] * dsub
    d_ref[...] = jnp.broadcast_to(dsub, (N_NODES, 8))


_R = 1000


def _scale_call(degp, x):
    return pl.pallas_call(
        _scale_body,
        out_shape=[
            jax.ShapeDtypeStruct((N_NODES, D), _f32),
            jax.ShapeDtypeStruct((N_NODES, 8), _f32),
        ],
        grid=(1,),
        in_specs=[
            pl.BlockSpec((NC, NBINS), lambda i: (0, 0)),
            pl.BlockSpec((N_NODES, D), lambda i: (0, 0)),
        ],
        out_specs=[
            pl.BlockSpec((N_NODES, D), lambda i: (0, 0)),
            pl.BlockSpec((N_NODES, 8), lambda i: (0, 0)),
        ],
    )(degp, x)


def _final_body(d_ref, acc_ref, y_ref, w_ref, b_ref, a_ref, o_ref):
    d = d_ref[:, 0:1]
    sfull = (acc_ref[0] + acc_ref[1] + y_ref[...]) * d
    z = jnp.dot(sfull, w_ref[...], preferred_element_type=_f32) + b_ref[...]
    o_ref[...] = jnp.where(z >= 0, z, a_ref[...] * z)


def _final_call(dcol, acc, y, W, b2, a2):
    return pl.pallas_call(
        _final_body,
        out_shape=jax.ShapeDtypeStruct((N_NODES, D), _f32),
        grid=(N_NODES // _R,),
        in_specs=[
            pl.BlockSpec((_R, 8), lambda i: (i, 0)),
            pl.BlockSpec((NC, _R, D), lambda i: (0, i, 0)),
            pl.BlockSpec((_R, D), lambda i: (i, 0)),
            pl.BlockSpec((D, D), lambda i: (0, 0)),
            pl.BlockSpec((1, D), lambda i: (0, 0)),
            pl.BlockSpec((1, D), lambda i: (0, 0)),
        ],
        out_specs=pl.BlockSpec((_R, D), lambda i: (i, 0)),
    )(dcol, acc, y, W, b2, a2)


def kernel(x, edge_index, W, b, alpha):
    ei = edge_index.astype(jnp.int32)
    dst1 = _split_call(ei, 1)
    src1 = _split_call(ei, 0)

    degp = _deg_call(dst1)
    y, dcol = _scale_call(degp, x)
    acc = _agg_call(y, src1, dst1)
    out = _final_call(dcol, acc, y, W,
                      b.reshape(1, D), alpha.reshape(1, D))
    return out
